# Initial kernel scaffold; baseline (speedup 1.0000x reference)
import jax
import jax.numpy as jnp
from jax.experimental import pallas as pl


def kernel(x, edge_index, S_edge_index, W0, b0, W1, b1, W2, b2, W3, b3, W4, b4, pW0, pb0, g0, be0, rm0, rv0, pW1, pb1, g1, be1, rm1, rv1):
    raise NotImplementedError("write your pallas kernel here")


# speedup vs baseline: 12.4373x; 12.4373x over previous
import functools

import jax
import jax.numpy as jnp
import numpy as np
from jax import lax
from jax.experimental import pallas as pl
from jax.experimental.pallas import tpu as pltpu
from jax.experimental.pallas import tpu_sc as plsc

_N = 10000
_NT = 10112
_D = 128
_E = 320000
_ES = 320000
_NW = 32
_JG = 82
_JS = 80
_JP = 242
_E2 = _NW * _JG * 128
_ES2 = _NW * _JS * 128
_RPT = _NT // 16

_mesh = plsc.VectorSubcoreMesh(core_axis_name="c", subcore_axis_name="s")


def _trash_np(n):
    return (10000 + (np.arange(n) % 112)).astype(np.int32)


def _make_rowpass(J, D):
    @functools.partial(
        pl.kernel,
        out_type=jax.ShapeDtypeStruct((2, _NT, D), jnp.float32),
        mesh=_mesh,
        scratch_types=[
            pltpu.VMEM((J, 128), jnp.int32),
            pltpu.VMEM((J, 128), jnp.int32),
            pltpu.VMEM((128, D), jnp.float32),
            pltpu.VMEM_SHARED((_NT, D), jnp.float32),
            pltpu.SemaphoreType.DMA,
        ],
    )
    def rp(table, gidx, sidx, out, gv, sv, rows, acc, sem):
        c = lax.axis_index("c")
        s = lax.axis_index("s")
        w = s * 2 + c
        z16 = jnp.zeros((16,), jnp.float32)

        def zrow(r, carry):
            for k in range(D // 16):
                rows[r, pl.ds(k * 16, 16)] = z16
            return carry

        lax.fori_loop(0, 128, zrow, 0)
        base = s * _RPT
        for k in range(4):
            pltpu.sync_copy(rows, acc.at[pl.ds(base + k * 128, 128)])
        pltpu.sync_copy(rows.at[pl.ds(0, _RPT - 512)],
                        acc.at[pl.ds(base + 512, _RPT - 512)])
        plsc.subcore_barrier()

        pltpu.sync_copy(gidx.at[w], gv)
        pltpu.sync_copy(sidx.at[w], sv)

        def step(j, carry):
            pltpu.async_copy(table.at[gv.at[j]], rows, sem).wait()
            pltpu.sync_copy(rows, acc.at[sv.at[j]], add=True)
            return carry

        lax.fori_loop(0, J, step, 0)
        plsc.subcore_barrier()
        for k in range(4):
            pltpu.sync_copy(acc.at[pl.ds(base + k * 128, 128)], rows)
            pltpu.sync_copy(rows, out.at[c, pl.ds(base + k * 128, 128)])
        pltpu.sync_copy(acc.at[pl.ds(base + 512, _RPT - 512)],
                        rows.at[pl.ds(0, _RPT - 512)])
        pltpu.sync_copy(rows.at[pl.ds(0, _RPT - 512)],
                        out.at[c, pl.ds(base + 512, _RPT - 512)])

    return rp


_ROWPASS_G = _make_rowpass(_JG, _D)
_ROWPASS_S = _make_rowpass(_JS, _D)

_P1S = _NT * 3 // 16


@functools.partial(
    pl.kernel,
    out_type=jax.ShapeDtypeStruct((2 * 3 * _NT,), jnp.float32),
    mesh=_mesh,
    scratch_types=[
        pltpu.VMEM((_JP, 128), jnp.int32),
        pltpu.VMEM((128,), jnp.float32),
        pltpu.VMEM((1904,), jnp.float32),
        pltpu.VMEM_SHARED((3 * _NT,), jnp.float32),
    ],
)
def _p1(fidx, out, iv, ones_v, zb, acc):
    c = lax.axis_index("c")
    s = lax.axis_index("s")
    w = s * 2 + c
    one16 = jnp.ones((16,), jnp.float32)
    z16 = jnp.zeros((16,), jnp.float32)
    for k in range(8):
        ones_v[pl.ds(k * 16, 16)] = one16

    def zfill(i, carry):
        zb[pl.ds(i * 16, 16)] = z16
        return carry

    lax.fori_loop(0, 119, zfill, 0)
    base = s * _P1S
    pltpu.sync_copy(zb.at[pl.ds(0, _P1S)], acc.at[pl.ds(base, _P1S)])
    plsc.subcore_barrier()
    pltpu.sync_copy(fidx.at[w], iv)

    def step(j, carry):
        pltpu.sync_copy(ones_v, acc.at[iv.at[j]], add=True)
        return carry

    lax.fori_loop(0, _JP, step, 0)
    plsc.subcore_barrier()
    pltpu.sync_copy(acc.at[pl.ds(base, _P1S)], zb.at[pl.ds(0, _P1S)])
    pltpu.sync_copy(zb.at[pl.ds(0, _P1S)],
                    out.at[pl.ds(c * (3 * _NT) + base, _P1S)])


def _rows_iota(shape):
    return (lax.broadcasted_iota(jnp.int32, shape, 0) * 128
            + lax.broadcasted_iota(jnp.int32, shape, 1))


def _q1k_body(p6, srcE, S0p, S1p, dinv_o, sizes_o, sepAS_o, sepBG_o,
              sepBS_o):
    p = p6[...]
    deg = p[0] + p[3]
    m0 = p[1] + p[4]
    m1 = p[2] + p[5]
    r = _rows_iota((79, 128))
    valid = r < _N
    dinv_o[...] = jnp.where(valid & (deg > 0.0), lax.rsqrt(deg), 0.0)
    size1 = jnp.sum(jnp.where(valid & (m0 > 0.0), 1, 0))
    size0 = jnp.sum(jnp.where(valid & (m1 > 0.0), 1, 0))
    i8 = lax.broadcasted_iota(jnp.int32, (8, 128), 0)
    sizes_o[...] = jnp.where(i8 == 0, size0, size1)
    s0 = S0p[...]
    s1 = S1p[...]
    tS = 10000 + _rows_iota(s0.shape) % 112
    sepAS_o[...] = jnp.where(s0 < size0, s0, tS)
    sepBG_o[...] = jnp.minimum(s0, size0 - 1)
    sepBS_o[...] = jnp.where(s1 < size1, s1, tS)


def _mask_rows(h):
    r = lax.broadcasted_iota(jnp.int32, (_NT, 1), 0)
    return jnp.where(r < _N, h, 0.0)


def _dot(a, b):
    return jnp.dot(a, b, preferred_element_type=jnp.float32,
                   precision=lax.Precision.HIGHEST)


def _t0_body(xp, dinvc, W, out):
    out[...] = dinvc[...] * _dot(xp[...], W[...])


def _ta_body(p2, dinvc, b, W, out):
    d = dinvc[...]
    h = jnp.maximum(d * (p2[0] + p2[1]) + b[...], 0.0)
    out[...] = d * _dot(_mask_rows(h), W[...])


def _t2_body(p2, dinvc, b, out):
    h = jnp.maximum(dinvc[...] * (p2[0] + p2[1]) + b[...], 0.0)
    out[...] = _mask_rows(h)


def _bn(z, g, be, rm, rv):
    return (z - rm[...]) * lax.rsqrt(rv[...] + 1e-5) * g[...] + be[...]


def _t3_body(p2, pW, pb, g, be, rm, rv, out):
    z = jnp.maximum(_dot(p2[0] + p2[1], pW[...]) + pb[...], 0.0)
    out[...] = _mask_rows(jnp.maximum(_bn(z, g, be, rm, rv), 0.0))


def _t4_body(p2, pW, pb, g, be, rm, rv, W, dinvc, sizes, out):
    z = jnp.maximum(_dot(p2[0] + p2[1], pW[...]) + pb[...], 0.0)
    h = _mask_rows(jnp.maximum(_bn(z, g, be, rm, rv), 0.0))
    hw = _dot(h, W[...])
    szrow = sizes[1:2, :]
    sel = lax.broadcasted_iota(jnp.int32, (_NT, _D), 0) == (szrow - 1)
    hwc = jnp.sum(jnp.where(sel, hw, 0.0), axis=0, keepdims=True)
    rcol = lax.broadcasted_iota(jnp.int32, (_NT, 1), 0)
    lim = sizes[1:2, 0:1]
    hw_clamped = jnp.where(rcol < lim, hw, jnp.broadcast_to(hwc, (_NT, _D)))
    out[...] = dinvc[...] * hw_clamped


def _t7_body(p2, dinvc, b, out):
    out[...] = dinvc[...] * (p2[0] + p2[1]) + b[...]


def _tcall(body, out_shape, *args):
    return pl.pallas_call(body, out_shape=out_shape)(*args)


_F32 = jnp.float32
_TBL = jax.ShapeDtypeStruct((_NT, _D), _F32)


def kernel(x, edge_index, S_edge_index, W0, b0, W1, b1, W2, b2, W3, b3, W4,
           b4, pW0, pb0, g0, be0, rm0, rv0, pW1, pb1, g1, be1, rm1, rv1):
    src = edge_index[0].astype(jnp.int32)
    dst = edge_index[1].astype(jnp.int32)
    S0 = S_edge_index[0].astype(jnp.int32)
    S1 = S_edge_index[1].astype(jnp.int32)
    loop = jnp.arange(_N, dtype=jnp.int32)
    padE = jnp.asarray(_trash_np(_E2 - _E - _N))
    padS = jnp.asarray(_trash_np(_ES2 - _ES))
    srcE = jnp.concatenate([src, loop, padE])
    dstE = jnp.concatenate([dst, loop, padE])
    S0p = jnp.concatenate([S0, padS])
    S1p = jnp.concatenate([S1, padS])
    fidx = jnp.concatenate([dstE, S0p + _NT, S1p + 2 * _NT])

    p3 = _p1(fidx.reshape(_NW, _JP, 128))
    dinv79, sizes8, sepAS, sepBG, sepBS = _tcall(
        _q1k_body,
        (jax.ShapeDtypeStruct((79, 128), _F32),
         jax.ShapeDtypeStruct((8, 128), jnp.int32),
         jax.ShapeDtypeStruct((_ES2 // 128, 128), jnp.int32),
         jax.ShapeDtypeStruct((_ES2 // 128, 128), jnp.int32),
         jax.ShapeDtypeStruct((_ES2 // 128, 128), jnp.int32)),
        p3.reshape(6, 79, 128), srcE.reshape(-1, 128), S0p.reshape(-1, 128),
        S1p.reshape(-1, 128))

    dinvc = dinv79.reshape(_NT, 1)

    srcE3 = srcE.reshape(_NW, _JG, 128)
    dstE3 = dstE.reshape(_NW, _JG, 128)
    S1p3 = S1p.reshape(_NW, _JS, 128)
    sepAS3 = sepAS.reshape(_NW, _JS, 128)
    sepBG3 = sepBG.reshape(_NW, _JS, 128)
    sepBS3 = sepBS.reshape(_NW, _JS, 128)

    b0r, b1r, b2r, b3r, b4r = (v.reshape(1, _D) for v in (b0, b1, b2, b3, b4))
    pb0r, g0r, be0r, rm0r, rv0r = (v.reshape(1, _D)
                                   for v in (pb0, g0, be0, rm0, rv0))
    pb1r, g1r, be1r, rm1r, rv1r = (v.reshape(1, _D)
                                   for v in (pb1, g1, be1, rm1, rv1))

    xp = jnp.concatenate([x, jnp.zeros((_NT - _N, _D), _F32)], 0)
    t = _tcall(_t0_body, _TBL, xp, dinvc, W0)
    p = _ROWPASS_G(t, srcE3, dstE3)
    t = _tcall(_ta_body, _TBL, p, dinvc, b0r, W1)
    p = _ROWPASS_G(t, srcE3, dstE3)
    t = _tcall(_t2_body, _TBL, p, dinvc, b1r)
    p = _ROWPASS_S(t, S1p3, sepAS3)
    t = _tcall(_t3_body, _TBL, p, pW0, pb0r, g0r, be0r, rm0r, rv0r)
    p = _ROWPASS_S(t, sepBG3, sepBS3)
    t = _tcall(_t4_body, _TBL,
               p, pW1, pb1r, g1r, be1r, rm1r, rv1r, W2, dinvc, sizes8)
    p = _ROWPASS_G(t, srcE3, dstE3)
    t = _tcall(_ta_body, _TBL, p, dinvc, b2r, W3)
    p = _ROWPASS_G(t, srcE3, dstE3)
    t = _tcall(_ta_body, _TBL, p, dinvc, b3r, W4)
    p = _ROWPASS_G(t, srcE3, dstE3)
    out = _tcall(_t7_body, _TBL, p, dinvc, b4r)
    return out[:_N]
